# baseline (speedup 1.0000x reference)
import functools

import jax
import jax.numpy as jnp
from jax import lax
from jax.experimental import pallas as pl
from jax.experimental.pallas import tpu as pltpu
from jax.experimental.pallas import tpu_sc as plsc

HEADS = 12
D_K = 64
D_V = 64
L = 2048
BUCKET = 64
N_BUCKETS = 32
N_CHUNKS = L // BUCKET
DM = HEADS * D_K
RB = 128
N_RB = L // RB
NEG_MASK = -1e15
NEG_DIAG = -1e5
SCCHUNK = 64


def _iota(shape, dim):
    return jax.lax.broadcasted_iota(jnp.int32, shape, dim).astype(jnp.float32)


def _dot(a, b, dims, prec):
    return jax.lax.dot_general(a, b, (dims, ((), ())),
                               preferred_element_type=jnp.float32,
                               precision=prec)


def _sort_body(r_ref, q_ref, cidx_ref, offs_row_ref, offs_col_ref,
               pos_ref, o_ref):
    ib = pl.program_id(0)
    ir = pl.program_id(1)
    nb = pl.num_programs(0)
    f32 = jnp.float32
    hi = jax.lax.Precision.HIGHEST

    r2 = r_ref[0]
    lane = _iota((RB, N_BUCKETS), 1)

    def hblk(a, carry):
        qb = q_ref[0, pl.ds(a * RB, RB), :]
        rotb = jax.lax.dot_general(qb, r2, ((((1,), (0,))), ((), ())),
                                   preferred_element_type=f32)
        rot = jnp.concatenate([rotb, -rotb], axis=1)
        mx = jnp.max(rot, axis=1, keepdims=True)
        amin = jnp.min(jnp.where(rot == mx, lane, 1e9), axis=1, keepdims=True)
        o_ref[pl.ds(a * RB, RB), :] = (lane == amin).astype(f32)
        return carry

    jax.lax.fori_loop(0, N_RB, hblk, 0)

    onehot = o_ref[:]
    cnt_row = jnp.sum(onehot, axis=0, keepdims=True)
    slt = (_iota((N_BUCKETS, N_BUCKETS), 0)
           < _iota((N_BUCKETS, N_BUCKETS), 1)).astype(f32)
    offs_row = _dot(cnt_row, slt, ((1,), (0,)), hi)
    offs_row_ref[0, 0] = offs_row
    ones_col = jnp.ones((L, 1), f32)
    cnt_col = _dot(onehot, ones_col, ((0,), (0,)), hi)
    offs_col_ref[0, 0] = _dot(slt, cnt_col, ((0,), (0,)), hi)

    t128 = (_iota((RB, RB), 1) < _iota((RB, RB), 0)).astype(f32)
    eye = (_iota((RB, RB), 0) == _iota((RB, RB), 1)).astype(f32)
    ones_row = jnp.ones((1, RB), f32)
    base = (ib * L).astype(f32)

    def blk(a, run):
        ob = o_ref[pl.ds(a * RB, RB), :]
        excl = _dot(t128, ob, ((1,), (0,)), jax.lax.Precision.DEFAULT) + run
        posb = jnp.sum((excl + offs_row) * ob, axis=1, keepdims=True)
        pos_ref[pl.ds(a * RB, RB), :] = posb
        rowv = _dot(ones_row, (posb + base) * eye, ((1,), (0,)), hi)
        cidx_ref[0, 0, pl.ds(a, 1), :] = rowv.astype(jnp.int32)
        return run + jnp.sum(ob, axis=0, keepdims=True)

    jax.lax.fori_loop(0, N_RB, blk, jnp.zeros((1, N_BUCKETS), f32))


def _attn_body(qc_ref, kc_ref, kp_ref, vc_ref, vp_ref,
               offs_row_ref, offs_col_ref, out_ref):
    ic = pl.program_id(1)
    f32 = jnp.float32

    qcur = qc_ref[0, 0]
    k3 = jnp.concatenate([kp_ref[0, 0, BUCKET:], kc_ref[0, 0]], axis=0)
    v3 = jnp.concatenate([vp_ref[0, 0, BUCKET:], vc_ref[0, 0]], axis=0)

    base = (ic * 2 * BUCKET).astype(f32)
    i128 = _iota((2 * BUCKET, 1), 0)
    hs_q = jnp.sum(((base + i128) >= offs_row_ref[0]).astype(f32),
                   axis=1, keepdims=True)
    j192 = _iota((1, 3 * BUCKET), 1)
    tk = base - BUCKET + j192
    tk = jnp.where(tk < 0, tk + L, tk)
    hs_k = jnp.sum((tk >= offs_col_ref[0]).astype(f32),
                   axis=0, keepdims=True)
    in_win = jnp.where(i128 < BUCKET, (j192 < 2 * BUCKET).astype(f32),
                       (j192 >= BUCKET).astype(f32))
    not_ok = 1.0 - (hs_q == hs_k).astype(f32) * in_win
    diag = (j192 == i128 + BUCKET).astype(f32)
    bias = not_ok * NEG_MASK + diag * NEG_DIAG

    df = jax.lax.Precision.DEFAULT
    probs = []
    for h in range(HEADS):
        sl = slice(h * D_K, (h + 1) * D_K)
        logits = _dot(qcur[:, sl], k3[:, sl], ((1,), (1,)), df) * (1.0 / 8.0)
        logits = logits + bias
        m = jnp.max(logits, axis=1, keepdims=True)
        e = jnp.exp(logits - m)
        probs.append(e * (1.0 / jnp.sum(e, axis=1, keepdims=True)))
    outs = [_dot(probs[h], v3[:, h * D_V:(h + 1) * D_V], ((1,), (0,)), df)
            for h in range(HEADS)]
    out_ref[0, 0] = jnp.concatenate(outs, axis=1)


def _make_sc_scatter(n_rows_total, nw):
    rows_per_w = n_rows_total // nw
    n_ch = rows_per_w // SCCHUNK
    mesh = plsc.VectorSubcoreMesh(core_axis_name="c", subcore_axis_name="s")

    @functools.partial(
        pl.kernel, mesh=mesh,
        out_type=[jax.ShapeDtypeStruct((n_rows_total, DM), jnp.float32)
                  for _ in range(3)],
        scratch_types=[
            pltpu.VMEM((n_ch, SCCHUNK), jnp.int32),
            pltpu.VMEM((SCCHUNK, DM), jnp.float32),
            pltpu.VMEM((SCCHUNK, DM), jnp.float32),
            pltpu.SemaphoreType.DMA,
            pltpu.SemaphoreType.DMA,
        ],
    )
    def sc_scatter(q_hbm, k_hbm, v_hbm, idx_hbm, qs_hbm, ks_hbm, vs_hbm,
                   idx_v, buf0, buf1, sem0, sem1):
        nc = lax.axis_index("c")
        ns = lax.axis_index("s")
        wid = ns * 2 + nc
        n_src = q_hbm.shape[0]
        pltpu.sync_copy(idx_hbm.at[wid], idx_v)
        bufs = (buf0, buf1)
        sems = (sem0, sem1)
        seq = [(tbl, dst, j)
               for tbl, dst in ((q_hbm, qs_hbm), (k_hbm, ks_hbm),
                                (v_hbm, vs_hbm))
               for j in range(n_ch)]
        pending = [None, None]
        for t, (tbl, dst, j) in enumerate(seq):
            bi = t % 2
            if pending[bi] is not None:
                pending[bi].wait()
            src_row = lax.rem(wid * rows_per_w + j * SCCHUNK, n_src)
            pltpu.sync_copy(tbl.at[pl.ds(src_row, SCCHUNK)], bufs[bi])
            pending[bi] = pltpu.async_copy(bufs[bi], dst.at[idx_v.at[j]],
                                           sems[bi])
        pending[0].wait()
        pending[1].wait()

    return sc_scatter


def _make_sc_combine(n_out_rows, nw):
    rows_per_w = n_out_rows // nw
    n_ch = rows_per_w // SCCHUNK
    mesh = plsc.VectorSubcoreMesh(core_axis_name="c", subcore_axis_name="s")

    @functools.partial(
        pl.kernel, mesh=mesh,
        out_type=jax.ShapeDtypeStruct((n_out_rows, DM), jnp.float32),
        scratch_types=[
            pltpu.VMEM((rows_per_w,), jnp.int32),
            pltpu.VMEM((rows_per_w,), jnp.int32),
            pltpu.VMEM((SCCHUNK, DM), jnp.float32),
            pltpu.VMEM((SCCHUNK, DM), jnp.float32),
            pltpu.SemaphoreType.DMA,
            pltpu.SemaphoreType.DMA,
        ],
    )
    def sc_combine(outs0_hbm, outs1_hbm, idx0_hbm, idx1_hbm, outf_hbm,
                   i0_v, i1_v, buf0, buf1, sem0, sem1):
        nc = lax.axis_index("c")
        ns = lax.axis_index("s")
        wid = ns * 2 + nc
        base = wid * rows_per_w
        pltpu.sync_copy(idx0_hbm.at[pl.ds(base, rows_per_w)], i0_v)
        pltpu.sync_copy(idx1_hbm.at[pl.ds(base, rows_per_w)], i1_v)
        for j in range(n_ch):
            c0 = pltpu.async_copy(
                outs0_hbm.at[i0_v.at[pl.ds(j * SCCHUNK, SCCHUNK)]], buf0, sem0)
            c1 = pltpu.async_copy(
                outs1_hbm.at[i1_v.at[pl.ds(j * SCCHUNK, SCCHUNK)]], buf1, sem1)
            c0.wait()
            c1.wait()

            def row_add(i, carry):
                for cc in range(DM // 16):
                    s = pl.ds(cc * 16, 16)
                    buf0[i, s] = buf0[i, s] + buf1[i, s]
                return carry

            lax.fori_loop(0, SCCHUNK, row_add, 0)
            pltpu.sync_copy(buf0, outf_hbm.at[pl.ds(base + j * SCCHUNK,
                                                    SCCHUNK)])

    return sc_combine


def kernel(q, k, v, R):
    b = q.shape[0]
    rnd = R.shape[1]
    rt = jnp.transpose(R, (1, 0, 2))
    f32 = jnp.float32
    i32 = jnp.int32

    cidx, offs_row, offs_col = pl.pallas_call(
        _sort_body,
        grid=(b, rnd),
        in_specs=[
            pl.BlockSpec((1, DM, N_BUCKETS // 2), lambda ib, ir: (ir, 0, 0)),
            pl.BlockSpec((1, L, DM), lambda ib, ir: (ib, 0, 0)),
        ],
        out_specs=[
            pl.BlockSpec((1, 1, N_RB, RB), lambda ib, ir: (ir, ib, 0, 0)),
            pl.BlockSpec((1, 1, 1, N_BUCKETS), lambda ib, ir: (ir, ib, 0, 0)),
            pl.BlockSpec((1, 1, N_BUCKETS, 1), lambda ib, ir: (ir, ib, 0, 0)),
        ],
        out_shape=[
            jax.ShapeDtypeStruct((rnd, b, N_RB, RB), i32),
            jax.ShapeDtypeStruct((rnd, b, 1, N_BUCKETS), f32),
            jax.ShapeDtypeStruct((rnd, b, N_BUCKETS, 1), f32),
        ],
        scratch_shapes=[
            pltpu.VMEM((L, 1), f32),
            pltpu.VMEM((L, N_BUCKETS), f32),
        ],
    )(rt, q)

    info = plsc.get_sparse_core_info()
    nw = info.num_cores * info.num_subcores
    nrows = b * L

    qt = q.reshape(nrows, DM)
    kt = k.reshape(nrows, DM)
    vt = v.reshape(nrows, DM)
    sc_scatter = _make_sc_scatter(nrows, nw)
    npair = N_CHUNKS // 2
    ch2 = 2 * BUCKET
    cur = lambda i, c: (i, c, 0, 0)
    prv = lambda i, c: (i, (c + npair - 1) % npair, 0, 0)
    cidx_r = cidx.reshape(rnd, nw, (nrows // nw) // SCCHUNK, SCCHUNK)
    orow = offs_row.reshape(rnd * b, 1, N_BUCKETS)
    ocol = offs_col.reshape(rnd * b, N_BUCKETS, 1)
    round_outs = []
    for r in range(rnd):
        qs, ks, vs = sc_scatter(qt, kt, vt, cidx_r[r])
        qs4 = qs.reshape(b, npair, ch2, DM)
        ks4 = ks.reshape(b, npair, ch2, DM)
        vs4 = vs.reshape(b, npair, ch2, DM)
        outs = pl.pallas_call(
            _attn_body,
            grid=(b, npair),
            in_specs=[
                pl.BlockSpec((1, 1, ch2, DM), cur),
                pl.BlockSpec((1, 1, ch2, DM), cur),
                pl.BlockSpec((1, 1, ch2, DM), prv),
                pl.BlockSpec((1, 1, ch2, DM), cur),
                pl.BlockSpec((1, 1, ch2, DM), prv),
                pl.BlockSpec((1, 1, N_BUCKETS), lambda i, c: (i, 0, 0)),
                pl.BlockSpec((1, N_BUCKETS, 1), lambda i, c: (i, 0, 0)),
            ],
            out_specs=pl.BlockSpec((1, 1, ch2, DM), cur),
            out_shape=jax.ShapeDtypeStruct((b, npair, ch2, DM), f32),
        )(qs4, ks4, ks4, vs4, vs4,
          lax.slice_in_dim(orow, r * b, (r + 1) * b),
          lax.slice_in_dim(ocol, r * b, (r + 1) * b))
        round_outs.append(outs.reshape(nrows, DM))

    cflat = cidx.reshape(rnd, nrows)
    out = _make_sc_combine(nrows, nw)(
        round_outs[0], round_outs[1], cflat[0], cflat[1])
    return out.reshape(b, L, HEADS, D_V)


# speedup vs baseline: 1.0256x; 1.0256x over previous
import functools

import jax
import jax.numpy as jnp
from jax import lax
from jax.experimental import pallas as pl
from jax.experimental.pallas import tpu as pltpu
from jax.experimental.pallas import tpu_sc as plsc

HEADS = 12
D_K = 64
D_V = 64
L = 2048
BUCKET = 64
N_BUCKETS = 32
N_CHUNKS = L // BUCKET
DM = HEADS * D_K
RB = 128
N_RB = L // RB
NEG_MASK = -1e15
NEG_DIAG = -1e5
SCCHUNK = 64


def _iota(shape, dim):
    return jax.lax.broadcasted_iota(jnp.int32, shape, dim).astype(jnp.float32)


def _dot(a, b, dims, prec):
    return jax.lax.dot_general(a, b, (dims, ((), ())),
                               preferred_element_type=jnp.float32,
                               precision=prec)


def _sort_body(r_ref, q_ref, cidx_ref, offs_row_ref, offs_col_ref,
               pos_ref, o_ref):
    ib = pl.program_id(0)
    ir = pl.program_id(1)
    nb = pl.num_programs(0)
    f32 = jnp.float32
    hi = jax.lax.Precision.HIGHEST

    r2 = r_ref[0]
    lane = _iota((RB, N_BUCKETS), 1)

    def hblk(a, carry):
        qb = q_ref[0, pl.ds(a * RB, RB), :]
        rotb = jax.lax.dot_general(qb, r2, ((((1,), (0,))), ((), ())),
                                   preferred_element_type=f32)
        rot = jnp.concatenate([rotb, -rotb], axis=1)
        mx = jnp.max(rot, axis=1, keepdims=True)
        amin = jnp.min(jnp.where(rot == mx, lane, 1e9), axis=1, keepdims=True)
        o_ref[pl.ds(a * RB, RB), :] = (lane == amin).astype(f32)
        return carry

    jax.lax.fori_loop(0, N_RB, hblk, 0)

    onehot = o_ref[:]
    cnt_row = jnp.sum(onehot, axis=0, keepdims=True)
    slt = (_iota((N_BUCKETS, N_BUCKETS), 0)
           < _iota((N_BUCKETS, N_BUCKETS), 1)).astype(f32)
    offs_row = _dot(cnt_row, slt, ((1,), (0,)), hi)
    offs_row_ref[0, 0] = offs_row
    ones_col = jnp.ones((L, 1), f32)
    cnt_col = _dot(onehot, ones_col, ((0,), (0,)), hi)
    offs_col_ref[0, 0] = _dot(slt, cnt_col, ((0,), (0,)), hi)

    t128 = (_iota((RB, RB), 1) < _iota((RB, RB), 0)).astype(f32)
    base = (ib * L).astype(f32)

    def blk(a, run):
        ob = o_ref[pl.ds(a * RB, RB), :]
        excl = _dot(t128, ob, ((1,), (0,)), jax.lax.Precision.DEFAULT) + run
        posb = jnp.sum((excl + offs_row) * ob, axis=1, keepdims=True)
        pos_ref[pl.ds(a * RB, RB), :] = posb
        cidx_ref[0, 0, pl.ds(a * RB, RB), :] = (posb + base).astype(jnp.int32)
        return run + jnp.sum(ob, axis=0, keepdims=True)

    jax.lax.fori_loop(0, N_RB, blk, jnp.zeros((1, N_BUCKETS), f32))


def _attn_body(qc_ref, kc_ref, kp_ref, vc_ref, vp_ref,
               offs_row_ref, offs_col_ref, out_ref):
    ic = pl.program_id(1)
    f32 = jnp.float32

    qcur = qc_ref[0, 0]
    k3 = jnp.concatenate([kp_ref[0, 0, BUCKET:], kc_ref[0, 0]], axis=0)
    v3 = jnp.concatenate([vp_ref[0, 0, BUCKET:], vc_ref[0, 0]], axis=0)

    base = (ic * 2 * BUCKET).astype(f32)
    i128 = _iota((2 * BUCKET, 1), 0)
    hs_q = jnp.sum(((base + i128) >= offs_row_ref[0]).astype(f32),
                   axis=1, keepdims=True)
    j192 = _iota((1, 3 * BUCKET), 1)
    tk = base - BUCKET + j192
    tk = jnp.where(tk < 0, tk + L, tk)
    hs_k = jnp.sum((tk >= offs_col_ref[0]).astype(f32),
                   axis=0, keepdims=True)
    in_win = jnp.where(i128 < BUCKET, (j192 < 2 * BUCKET).astype(f32),
                       (j192 >= BUCKET).astype(f32))
    not_ok = 1.0 - (hs_q == hs_k).astype(f32) * in_win
    diag = (j192 == i128 + BUCKET).astype(f32)
    bias = not_ok * NEG_MASK + diag * NEG_DIAG

    df = jax.lax.Precision.DEFAULT
    probs = []
    for h in range(HEADS):
        sl = slice(h * D_K, (h + 1) * D_K)
        logits = _dot(qcur[:, sl], k3[:, sl], ((1,), (1,)), df) * (1.0 / 8.0)
        logits = logits + bias
        m = jnp.max(logits, axis=1, keepdims=True)
        e = jnp.exp(logits - m)
        probs.append(e * (1.0 / jnp.sum(e, axis=1, keepdims=True)))
    outs = [_dot(probs[h], v3[:, h * D_V:(h + 1) * D_V], ((1,), (0,)), df)
            for h in range(HEADS)]
    out_ref[0, 0] = jnp.concatenate(outs, axis=1)


def _make_sc_scatter(n_rows_total, nw):
    rows_per_w = n_rows_total // nw
    n_ch = rows_per_w // SCCHUNK
    mesh = plsc.VectorSubcoreMesh(core_axis_name="c", subcore_axis_name="s")

    @functools.partial(
        pl.kernel, mesh=mesh,
        out_type=[jax.ShapeDtypeStruct((n_rows_total, DM), jnp.float32)
                  for _ in range(3)],
        scratch_types=[
            pltpu.VMEM((n_ch, SCCHUNK), jnp.int32),
            pltpu.VMEM((SCCHUNK, DM), jnp.float32),
            pltpu.VMEM((SCCHUNK, DM), jnp.float32),
            pltpu.SemaphoreType.DMA,
            pltpu.SemaphoreType.DMA,
        ],
    )
    def sc_scatter(q_hbm, k_hbm, v_hbm, idx_hbm, qs_hbm, ks_hbm, vs_hbm,
                   idx_v, buf0, buf1, sem0, sem1):
        nc = lax.axis_index("c")
        ns = lax.axis_index("s")
        wid = ns * 2 + nc
        n_src = q_hbm.shape[0]
        pltpu.sync_copy(idx_hbm.at[wid], idx_v)
        bufs = (buf0, buf1)
        sems = (sem0, sem1)
        seq = [(tbl, dst, j)
               for tbl, dst in ((q_hbm, qs_hbm), (k_hbm, ks_hbm),
                                (v_hbm, vs_hbm))
               for j in range(n_ch)]
        pending = [None, None]
        for t, (tbl, dst, j) in enumerate(seq):
            bi = t % 2
            if pending[bi] is not None:
                pending[bi].wait()
            src_row = lax.rem(wid * rows_per_w + j * SCCHUNK, n_src)
            pltpu.sync_copy(tbl.at[pl.ds(src_row, SCCHUNK)], bufs[bi])
            pending[bi] = pltpu.async_copy(bufs[bi], dst.at[idx_v.at[j]],
                                           sems[bi])
        pending[0].wait()
        pending[1].wait()

    return sc_scatter


def _make_sc_combine(n_out_rows, nw):
    rows_per_w = n_out_rows // nw
    n_ch = rows_per_w // SCCHUNK
    mesh = plsc.VectorSubcoreMesh(core_axis_name="c", subcore_axis_name="s")

    @functools.partial(
        pl.kernel, mesh=mesh,
        out_type=jax.ShapeDtypeStruct((n_out_rows, DM), jnp.float32),
        scratch_types=[
            pltpu.VMEM((rows_per_w,), jnp.int32),
            pltpu.VMEM((rows_per_w,), jnp.int32),
            pltpu.VMEM((SCCHUNK, DM), jnp.float32),
            pltpu.VMEM((SCCHUNK, DM), jnp.float32),
            pltpu.SemaphoreType.DMA,
            pltpu.SemaphoreType.DMA,
        ],
    )
    def sc_combine(outs0_hbm, outs1_hbm, idx0_hbm, idx1_hbm, outf_hbm,
                   i0_v, i1_v, buf0, buf1, sem0, sem1):
        nc = lax.axis_index("c")
        ns = lax.axis_index("s")
        wid = ns * 2 + nc
        base = wid * rows_per_w
        pltpu.sync_copy(idx0_hbm.at[pl.ds(base, rows_per_w)], i0_v)
        pltpu.sync_copy(idx1_hbm.at[pl.ds(base, rows_per_w)], i1_v)
        for j in range(n_ch):
            c0 = pltpu.async_copy(
                outs0_hbm.at[i0_v.at[pl.ds(j * SCCHUNK, SCCHUNK)]], buf0, sem0)
            c1 = pltpu.async_copy(
                outs1_hbm.at[i1_v.at[pl.ds(j * SCCHUNK, SCCHUNK)]], buf1, sem1)
            c0.wait()
            c1.wait()

            def row_add(i, carry):
                for cc in range(DM // 16):
                    s = pl.ds(cc * 16, 16)
                    buf0[i, s] = buf0[i, s] + buf1[i, s]
                return carry

            lax.fori_loop(0, SCCHUNK, row_add, 0)
            pltpu.sync_copy(buf0, outf_hbm.at[pl.ds(base + j * SCCHUNK,
                                                    SCCHUNK)])

    return sc_combine


def kernel(q, k, v, R):
    b = q.shape[0]
    rnd = R.shape[1]
    rt = jnp.transpose(R, (1, 0, 2))
    f32 = jnp.float32
    i32 = jnp.int32

    cidx, offs_row, offs_col = pl.pallas_call(
        _sort_body,
        grid=(b, rnd),
        in_specs=[
            pl.BlockSpec((1, DM, N_BUCKETS // 2), lambda ib, ir: (ir, 0, 0)),
            pl.BlockSpec((1, L, DM), lambda ib, ir: (ib, 0, 0)),
        ],
        out_specs=[
            pl.BlockSpec((1, 1, L, 1), lambda ib, ir: (ir, ib, 0, 0)),
            pl.BlockSpec((1, 1, 1, N_BUCKETS), lambda ib, ir: (ir, ib, 0, 0)),
            pl.BlockSpec((1, 1, N_BUCKETS, 1), lambda ib, ir: (ir, ib, 0, 0)),
        ],
        out_shape=[
            jax.ShapeDtypeStruct((rnd, b, L, 1), i32),
            jax.ShapeDtypeStruct((rnd, b, 1, N_BUCKETS), f32),
            jax.ShapeDtypeStruct((rnd, b, N_BUCKETS, 1), f32),
        ],
        scratch_shapes=[
            pltpu.VMEM((L, 1), f32),
            pltpu.VMEM((L, N_BUCKETS), f32),
        ],
    )(rt, q)

    info = plsc.get_sparse_core_info()
    nw = info.num_cores * info.num_subcores
    nrows = b * L

    qt = q.reshape(nrows, DM)
    kt = k.reshape(nrows, DM)
    vt = v.reshape(nrows, DM)
    sc_scatter = _make_sc_scatter(nrows, nw)
    npair = N_CHUNKS // 2
    ch2 = 2 * BUCKET
    cur = lambda i, c: (i, c, 0, 0)
    prv = lambda i, c: (i, (c + npair - 1) % npair, 0, 0)
    cidx_r = cidx.reshape(rnd, nw, (nrows // nw) // SCCHUNK, SCCHUNK)
    orow = offs_row.reshape(rnd * b, 1, N_BUCKETS)
    ocol = offs_col.reshape(rnd * b, N_BUCKETS, 1)
    round_outs = []
    for r in range(rnd):
        qs, ks, vs = sc_scatter(qt, kt, vt, cidx_r[r])
        qs4 = qs.reshape(b, npair, ch2, DM)
        ks4 = ks.reshape(b, npair, ch2, DM)
        vs4 = vs.reshape(b, npair, ch2, DM)
        outs = pl.pallas_call(
            _attn_body,
            grid=(b, npair),
            in_specs=[
                pl.BlockSpec((1, 1, ch2, DM), cur),
                pl.BlockSpec((1, 1, ch2, DM), cur),
                pl.BlockSpec((1, 1, ch2, DM), prv),
                pl.BlockSpec((1, 1, ch2, DM), cur),
                pl.BlockSpec((1, 1, ch2, DM), prv),
                pl.BlockSpec((1, 1, N_BUCKETS), lambda i, c: (i, 0, 0)),
                pl.BlockSpec((1, N_BUCKETS, 1), lambda i, c: (i, 0, 0)),
            ],
            out_specs=pl.BlockSpec((1, 1, ch2, DM), cur),
            out_shape=jax.ShapeDtypeStruct((b, npair, ch2, DM), f32),
        )(qs4, ks4, ks4, vs4, vs4,
          lax.slice_in_dim(orow, r * b, (r + 1) * b),
          lax.slice_in_dim(ocol, r * b, (r + 1) * b))
        round_outs.append(outs.reshape(nrows, DM))

    cflat = cidx.reshape(rnd, nrows)
    out = _make_sc_combine(nrows, nw)(
        round_outs[0], round_outs[1], cflat[0], cflat[1])
    return out.reshape(b, L, HEADS, D_V)
